# baseline (speedup 1.0000x reference)
import jax
import jax.numpy as jnp
from jax import lax
from jax.experimental import pallas as pl
from jax.experimental.pallas import tpu as pltpu
from jax.experimental.pallas import tpu_sc as plsc

E = 3200000
N = 100000
NB = 4
NP = 100096
CHUNK = 512
NCHUNK = E // CHUNK
GROUPS = CHUNK // 16
ROWS_T = NP // 16

_f32 = jnp.float32
_i32 = jnp.int32


def _sc_body(evt, eidx, w2r, z8r, f_out, v_out,
             xb0, yb0, zb0, sidx0, didx0, fpos0, fneg0, vbuf0,
             xb1, yb1, zb1, sidx1, didx1, fpos1, fneg1, vbuf1,
             w2b, facc, vacc, in_sem0, in_sem1, sc_sem0, sc_sem1):
    cid = lax.axis_index("c")
    sid = lax.axis_index("s")
    wid = sid * 2 + cid

    r0 = sid * ROWS_T
    pltpu.sync_copy(z8r, facc.at[pl.ds(r0, ROWS_T)])
    pltpu.sync_copy(z8r, vacc.at[pl.ds(r0, ROWS_T)])
    pltpu.sync_copy(w2r, w2b)
    plsc.subcore_barrier()

    iota = lax.iota(_i32, 16)
    w2x = w2b[pl.ds(0, 16)]
    w2y = w2b[pl.ds(16, 16)]
    w2z = w2b[pl.ds(32, 16)]
    cols = [jnp.full((16,), c, _i32) for c in range(6)]

    base_pair = wid * 97 + jnp.minimum(wid, 21)
    npairs = 97 + (wid < 21).astype(_i32)

    sets = (
        (xb0, yb0, zb0, sidx0, didx0, fpos0, fneg0, vbuf0, in_sem0, sc_sem0),
        (xb1, yb1, zb1, sidx1, didx1, fpos1, fneg1, vbuf1, in_sem1, sc_sem1),
    )

    def issue_inputs(c, p):
        xb, yb, zb, sidx, didx, _, _, _, in_sem, _ = sets[p]
        e0 = c * CHUNK
        ds = pl.ds(e0, CHUNK)
        out = [
            pltpu.async_copy(evt.at[0].at[ds], xb, in_sem),
            pltpu.async_copy(evt.at[1].at[ds], yb, in_sem),
            pltpu.async_copy(evt.at[2].at[ds], zb, in_sem),
        ]
        b0 = c * (CHUNK // 128)
        for j in range(4):
            out.append(pltpu.async_copy(
                eidx.at[b0 + j].at[0], sidx.at[j], in_sem))
            out.append(pltpu.async_copy(
                eidx.at[b0 + j].at[1], didx.at[j], in_sem))
        return out

    def compute(p):
        xb, yb, zb, _, _, fpos, fneg, vbuf, _, _ = sets[p]

        def gbody(g, carry):
            rows = iota + g * 16
            x = xb[pl.ds(g * 16, 16)]
            y = yb[pl.ds(g * 16, 16)]
            z = zb[pl.ds(g * 16, 16)]
            s = x * x * w2x + y * y * w2y + z * z * w2z
            m = jnp.exp(-s) * (-2.0)
            fx = m * w2x * x
            fy = m * w2y * y
            fz = m * w2z * z
            plsc.store_scatter(fpos, [rows, cols[0]], fx)
            plsc.store_scatter(fpos, [rows, cols[1]], fy)
            plsc.store_scatter(fpos, [rows, cols[2]], fz)
            plsc.store_scatter(fneg, [rows, cols[0]], -fx)
            plsc.store_scatter(fneg, [rows, cols[1]], -fy)
            plsc.store_scatter(fneg, [rows, cols[2]], -fz)
            plsc.store_scatter(vbuf, [rows, cols[0]], x * fx)
            plsc.store_scatter(vbuf, [rows, cols[1]], y * fy)
            plsc.store_scatter(vbuf, [rows, cols[2]], z * fz)
            plsc.store_scatter(vbuf, [rows, cols[3]], x * fy)
            plsc.store_scatter(vbuf, [rows, cols[4]], y * fz)
            plsc.store_scatter(vbuf, [rows, cols[5]], z * fx)
            return carry

        lax.fori_loop(0, GROUPS, gbody, 0)

    def fire_scatters(p):
        _, _, _, sidx, didx, fpos, fneg, vbuf, _, sc_sem = sets[p]
        for j in range(4):
            rs = pl.ds(j * 128, 128)
            pltpu.async_copy(fpos.at[rs], facc.at[sidx.at[j]], sc_sem,
                             add=True)
            pltpu.async_copy(fneg.at[rs], facc.at[didx.at[j]], sc_sem,
                             add=True)
            pltpu.async_copy(vbuf.at[rs], vacc.at[didx.at[j]], sc_sem,
                             add=True)

    def drain_scatters(p):
        _, _, _, sidx, didx, fpos, fneg, vbuf, _, sc_sem = sets[p]
        for j in range(4):
            rs = pl.ds(j * 128, 128)
            pltpu.make_async_copy(
                fpos.at[rs], facc.at[sidx.at[j]], sc_sem).wait()
            pltpu.make_async_copy(
                fneg.at[rs], facc.at[didx.at[j]], sc_sem).wait()
            pltpu.make_async_copy(
                vbuf.at[rs], vacc.at[didx.at[j]], sc_sem).wait()

    def pair_body(q, carry):
        c0 = 2 * (base_pair + q)
        @pl.when(q > 0)
        def _():
            drain_scatters(0)
        in0 = issue_inputs(c0, 0)
        @pl.when(q > 0)
        def _():
            drain_scatters(1)
        in1 = issue_inputs(c0 + 1, 1)
        for d in in0:
            d.wait()
        compute(0)
        fire_scatters(0)
        for d in in1:
            d.wait()
        compute(1)
        fire_scatters(1)
        return carry

    lax.fori_loop(0, npairs, pair_body, 0)

    drain_scatters(0)
    drain_scatters(1)
    plsc.subcore_barrier()

    rows = pl.ds(r0, ROWS_T)
    pltpu.sync_copy(facc.at[rows], f_out.at[cid].at[rows])
    pltpu.sync_copy(vacc.at[rows], v_out.at[cid].at[rows])


@jax.jit
def _sc_scatter(evt, eidx, w2r, z8r):
    mesh = plsc.VectorSubcoreMesh(core_axis_name="c", subcore_axis_name="s")
    return pl.kernel(
        _sc_body,
        out_type=(
            jax.ShapeDtypeStruct((2, NP, 8), _f32),
            jax.ShapeDtypeStruct((2, NP, 8), _f32),
        ),
        mesh=mesh,
        scratch_types=(
            [pltpu.VMEM((CHUNK,), _f32)] * 3
            + [pltpu.VMEM((4, 128), _i32)] * 2
            + [pltpu.VMEM((CHUNK, 8), _f32)] * 3
            + [pltpu.VMEM((CHUNK,), _f32)] * 3
            + [pltpu.VMEM((4, 128), _i32)] * 2
            + [pltpu.VMEM((CHUNK, 8), _f32)] * 3
            + [
                pltpu.VMEM((48,), _f32),
                pltpu.VMEM_SHARED((NP, 8), _f32),
                pltpu.VMEM_SHARED((NP, 8), _f32),
                pltpu.SemaphoreType.DMA,
                pltpu.SemaphoreType.DMA,
                pltpu.SemaphoreType.DMA,
                pltpu.SemaphoreType.DMA,
            ]
        ),
        compiler_params=pltpu.CompilerParams(
            needs_layout_passes=False, use_tc_tiling_on_sc=False),
    )(evt, eidx, w2r, z8r)


BN = 4000
NSTEP = N // BN


def _tc_body(num_ref, vol_ref, batch_ref, f_ref, v_ref,
             force_ref, stress_ref, acc_ref):
    i = pl.program_id(0)
    one = (num_ref[0] - N + 1).astype(_f32)
    force_ref[...] = (f_ref[0, :, :3] + f_ref[1, :, :3]) * one
    v = v_ref[0, :, :6] + v_ref[1, :, :6]
    b = batch_ref[0]
    bid = lax.broadcasted_iota(_i32, (NB, 1), 0)
    onehot = (b == bid).astype(_f32)
    contrib = lax.dot_general(onehot, v, (((1,), (0,)), ((), ())),
                              preferred_element_type=_f32)

    @pl.when(i == 0)
    def _():
        acc_ref[...] = jnp.zeros((NB, 6), _f32)

    acc_ref[...] += contrib

    @pl.when(i == NSTEP - 1)
    def _():
        stress_ref[...] = -acc_ref[...] / vol_ref[...]


@jax.jit
def _tc_finalize(num_atoms, vol_r, batch_r, fpart, vpart):
    return pl.pallas_call(
        _tc_body,
        grid=(NSTEP,),
        in_specs=[
            pl.BlockSpec(memory_space=pltpu.SMEM),
            pl.BlockSpec((NB, 1), lambda i: (0, 0)),
            pl.BlockSpec((1, 1, BN), lambda i: (i, 0, 0)),
            pl.BlockSpec((2, BN, 8), lambda i: (0, i, 0)),
            pl.BlockSpec((2, BN, 8), lambda i: (0, i, 0)),
        ],
        out_specs=[
            pl.BlockSpec((BN, 3), lambda i: (i, 0)),
            pl.BlockSpec((NB, 6), lambda i: (0, 0)),
        ],
        out_shape=[
            jax.ShapeDtypeStruct((N, 3), _f32),
            jax.ShapeDtypeStruct((NB, 6), _f32),
        ],
        scratch_shapes=[pltpu.VMEM((NB, 6), _f32)],
        compiler_params=pltpu.CompilerParams(
            dimension_semantics=("arbitrary",)),
    )(num_atoms, vol_r, batch_r, fpart, vpart)


def kernel(edge_vec, edge_idx, num_atoms, batch, cell_volume, W):
    evt = edge_vec.T
    w2r = jnp.broadcast_to((W * W)[:, None], (3, 16)).reshape(48)
    z8r = jnp.zeros((ROWS_T, 8), _f32)
    eidx_r = edge_idx.reshape(2, E // 128, 128).transpose(1, 0, 2)
    fpart, vpart = _sc_scatter(evt, eidx_r, w2r, z8r)
    batch_r = batch.reshape(NSTEP, 1, BN)
    vol_r = cell_volume.reshape(NB, 1)
    force, stress = _tc_finalize(num_atoms, vol_r, batch_r, fpart, vpart)
    return force, stress


# speedup vs baseline: 1.8783x; 1.8783x over previous
import jax
import jax.numpy as jnp
from jax import lax
from jax.experimental import pallas as pl
from jax.experimental.pallas import tpu as pltpu
from jax.experimental.pallas import tpu_sc as plsc

E = 3200000
N = 100000
NB = 4
NP = 100096
CHUNK = 512
NCHUNK = E // CHUNK
GROUPS = CHUNK // 16
ROWS_T = NP // 16

_f32 = jnp.float32
_i32 = jnp.int32


def _sc_body(evx, evy, evz, srcr, dstr, w2r, z8r, f_out, v_out,
             xb0, yb0, zb0, sidx0, didx0, fpos0, fneg0, vbuf0,
             xb1, yb1, zb1, sidx1, didx1, fpos1, fneg1, vbuf1,
             w2b, facc, vacc, in_sem0, in_sem1, sc_sem0, sc_sem1):
    cid = lax.axis_index("c")
    sid = lax.axis_index("s")
    wid = sid * 2 + cid

    r0 = sid * ROWS_T
    pltpu.sync_copy(z8r, facc.at[pl.ds(r0, ROWS_T)])
    pltpu.sync_copy(z8r, vacc.at[pl.ds(r0, ROWS_T)])
    pltpu.sync_copy(w2r, w2b)
    plsc.subcore_barrier()

    iota = lax.iota(_i32, 16)
    w2x = w2b[pl.ds(0, 16)]
    w2y = w2b[pl.ds(16, 16)]
    w2z = w2b[pl.ds(32, 16)]
    cols = [jnp.full((16,), c, _i32) for c in range(6)]

    base_pair = wid * 97 + jnp.minimum(wid, 21)
    npairs = 97 + (wid < 21).astype(_i32)

    sets = (
        (xb0, yb0, zb0, sidx0, didx0, fpos0, fneg0, vbuf0, in_sem0, sc_sem0),
        (xb1, yb1, zb1, sidx1, didx1, fpos1, fneg1, vbuf1, in_sem1, sc_sem1),
    )

    def issue_inputs(c, p):
        xb, yb, zb, sidx, didx, _, _, _, in_sem, _ = sets[p]
        e0 = c * CHUNK
        return [
            pltpu.async_copy(evx.at[pl.ds(e0, CHUNK)], xb, in_sem),
            pltpu.async_copy(evy.at[pl.ds(e0, CHUNK)], yb, in_sem),
            pltpu.async_copy(evz.at[pl.ds(e0, CHUNK)], zb, in_sem),
            pltpu.async_copy(srcr.at[c], sidx, in_sem),
            pltpu.async_copy(dstr.at[c], didx, in_sem),
        ]

    def compute(p):
        xb, yb, zb, _, _, fpos, fneg, vbuf, _, _ = sets[p]

        def gbody(g, carry):
            rows = iota + g * 16
            x = xb[pl.ds(g * 16, 16)]
            y = yb[pl.ds(g * 16, 16)]
            z = zb[pl.ds(g * 16, 16)]
            s = x * x * w2x + y * y * w2y + z * z * w2z
            m = jnp.exp(-s) * (-2.0)
            fx = m * w2x * x
            fy = m * w2y * y
            fz = m * w2z * z
            plsc.store_scatter(fpos, [rows, cols[0]], fx)
            plsc.store_scatter(fpos, [rows, cols[1]], fy)
            plsc.store_scatter(fpos, [rows, cols[2]], fz)
            plsc.store_scatter(fneg, [rows, cols[0]], -fx)
            plsc.store_scatter(fneg, [rows, cols[1]], -fy)
            plsc.store_scatter(fneg, [rows, cols[2]], -fz)
            plsc.store_scatter(vbuf, [rows, cols[0]], x * fx)
            plsc.store_scatter(vbuf, [rows, cols[1]], y * fy)
            plsc.store_scatter(vbuf, [rows, cols[2]], z * fz)
            plsc.store_scatter(vbuf, [rows, cols[3]], x * fy)
            plsc.store_scatter(vbuf, [rows, cols[4]], y * fz)
            plsc.store_scatter(vbuf, [rows, cols[5]], z * fx)
            return carry

        lax.fori_loop(0, GROUPS, gbody, 0)

    def fire_scatters(p):
        _, _, _, sidx, didx, fpos, fneg, vbuf, _, sc_sem = sets[p]
        for j in range(4):
            rs = pl.ds(j * 128, 128)
            pltpu.async_copy(fpos.at[rs], facc.at[sidx.at[j]], sc_sem,
                             add=True)
            pltpu.async_copy(fneg.at[rs], facc.at[didx.at[j]], sc_sem,
                             add=True)
            pltpu.async_copy(vbuf.at[rs], vacc.at[didx.at[j]], sc_sem,
                             add=True)

    def drain_scatters(p):
        _, _, _, sidx, didx, fpos, fneg, vbuf, _, sc_sem = sets[p]
        for j in range(4):
            rs = pl.ds(j * 128, 128)
            pltpu.make_async_copy(
                fpos.at[rs], facc.at[sidx.at[j]], sc_sem).wait()
            pltpu.make_async_copy(
                fneg.at[rs], facc.at[didx.at[j]], sc_sem).wait()
            pltpu.make_async_copy(
                vbuf.at[rs], vacc.at[didx.at[j]], sc_sem).wait()

    def pair_body(q, carry):
        c0 = 2 * (base_pair + q)
        @pl.when(q > 0)
        def _():
            drain_scatters(0)
        in0 = issue_inputs(c0, 0)
        @pl.when(q > 0)
        def _():
            drain_scatters(1)
        in1 = issue_inputs(c0 + 1, 1)
        for d in in0:
            d.wait()
        compute(0)
        fire_scatters(0)
        for d in in1:
            d.wait()
        compute(1)
        fire_scatters(1)
        return carry

    lax.fori_loop(0, npairs, pair_body, 0)

    drain_scatters(0)
    drain_scatters(1)
    plsc.subcore_barrier()

    rows = pl.ds(r0, ROWS_T)
    pltpu.sync_copy(facc.at[rows], f_out.at[cid].at[rows])
    pltpu.sync_copy(vacc.at[rows], v_out.at[cid].at[rows])


@jax.jit
def _sc_scatter(evx, evy, evz, srcr, dstr, w2r, z8r):
    mesh = plsc.VectorSubcoreMesh(core_axis_name="c", subcore_axis_name="s")
    return pl.kernel(
        _sc_body,
        out_type=(
            jax.ShapeDtypeStruct((2, NP, 8), _f32),
            jax.ShapeDtypeStruct((2, NP, 8), _f32),
        ),
        mesh=mesh,
        scratch_types=(
            [pltpu.VMEM((CHUNK,), _f32)] * 3
            + [pltpu.VMEM((4, 128), _i32)] * 2
            + [pltpu.VMEM((CHUNK, 8), _f32)] * 3
            + [pltpu.VMEM((CHUNK,), _f32)] * 3
            + [pltpu.VMEM((4, 128), _i32)] * 2
            + [pltpu.VMEM((CHUNK, 8), _f32)] * 3
            + [
                pltpu.VMEM((48,), _f32),
                pltpu.VMEM_SHARED((NP, 8), _f32),
                pltpu.VMEM_SHARED((NP, 8), _f32),
                pltpu.SemaphoreType.DMA,
                pltpu.SemaphoreType.DMA,
                pltpu.SemaphoreType.DMA,
                pltpu.SemaphoreType.DMA,
            ]
        ),
        compiler_params=pltpu.CompilerParams(
            needs_layout_passes=False, use_tc_tiling_on_sc=False),
    )(evx, evy, evz, srcr, dstr, w2r, z8r)


BN = 4000
NSTEP = N // BN


def _tc_body(num_ref, vol_ref, batch_ref, f_ref, v_ref,
             force_ref, stress_ref, acc_ref):
    i = pl.program_id(0)
    one = (num_ref[0] - N + 1).astype(_f32)
    force_ref[...] = (f_ref[0, :, :3] + f_ref[1, :, :3]) * one
    v = v_ref[0, :, :6] + v_ref[1, :, :6]
    b = batch_ref[0]
    bid = lax.broadcasted_iota(_i32, (NB, 1), 0)
    onehot = (b == bid).astype(_f32)
    contrib = lax.dot_general(onehot, v, (((1,), (0,)), ((), ())),
                              preferred_element_type=_f32)

    @pl.when(i == 0)
    def _():
        acc_ref[...] = jnp.zeros((NB, 6), _f32)

    acc_ref[...] += contrib

    @pl.when(i == NSTEP - 1)
    def _():
        stress_ref[...] = -acc_ref[...] / vol_ref[...]


@jax.jit
def _tc_finalize(num_atoms, vol_r, batch_r, fpart, vpart):
    return pl.pallas_call(
        _tc_body,
        grid=(NSTEP,),
        in_specs=[
            pl.BlockSpec(memory_space=pltpu.SMEM),
            pl.BlockSpec((NB, 1), lambda i: (0, 0)),
            pl.BlockSpec((1, 1, BN), lambda i: (i, 0, 0)),
            pl.BlockSpec((2, BN, 8), lambda i: (0, i, 0)),
            pl.BlockSpec((2, BN, 8), lambda i: (0, i, 0)),
        ],
        out_specs=[
            pl.BlockSpec((BN, 3), lambda i: (i, 0)),
            pl.BlockSpec((NB, 6), lambda i: (0, 0)),
        ],
        out_shape=[
            jax.ShapeDtypeStruct((N, 3), _f32),
            jax.ShapeDtypeStruct((NB, 6), _f32),
        ],
        scratch_shapes=[pltpu.VMEM((NB, 6), _f32)],
        compiler_params=pltpu.CompilerParams(
            dimension_semantics=("arbitrary",)),
    )(num_atoms, vol_r, batch_r, fpart, vpart)


def kernel(edge_vec, edge_idx, num_atoms, batch, cell_volume, W):
    evx = edge_vec[:, 0]
    evy = edge_vec[:, 1]
    evz = edge_vec[:, 2]
    srcr = edge_idx[0].reshape(NCHUNK, 4, 128)
    dstr = edge_idx[1].reshape(NCHUNK, 4, 128)
    w2r = jnp.broadcast_to((W * W)[:, None], (3, 16)).reshape(48)
    z8r = jnp.zeros((ROWS_T, 8), _f32)
    fpart, vpart = _sc_scatter(evx, evy, evz, srcr, dstr, w2r, z8r)
    batch_r = batch.reshape(NSTEP, 1, BN)
    vol_r = cell_volume.reshape(NB, 1)
    force, stress = _tc_finalize(num_atoms, vol_r, batch_r, fpart, vpart)
    return force, stress


# speedup vs baseline: 1.9671x; 1.0473x over previous
import jax
import jax.numpy as jnp
from jax import lax
from jax.experimental import pallas as pl
from jax.experimental.pallas import tpu as pltpu
from jax.experimental.pallas import tpu_sc as plsc

E = 3200000
N = 100000
NB = 4
NP = 100096
CHUNK = 512
NCHUNK = E // CHUNK
GROUPS = CHUNK // 16
ROWS_T = NP // 16

_f32 = jnp.float32
_i32 = jnp.int32


def _sc_body(evx, evy, evz, eidx, w2r, z8r, f_out, v_out,
             xb0, yb0, zb0, sidx0, didx0, fpos0, fneg0, vbuf0,
             xb1, yb1, zb1, sidx1, didx1, fpos1, fneg1, vbuf1,
             w2b, facc, vacc, in_sem0, in_sem1, sc_sem0, sc_sem1):
    cid = lax.axis_index("c")
    sid = lax.axis_index("s")
    wid = sid * 2 + cid

    r0 = sid * ROWS_T
    pltpu.sync_copy(z8r, facc.at[pl.ds(r0, ROWS_T)])
    pltpu.sync_copy(z8r, vacc.at[pl.ds(r0, ROWS_T)])
    pltpu.sync_copy(w2r, w2b)
    plsc.subcore_barrier()

    iota = lax.iota(_i32, 16)
    w2x = w2b[pl.ds(0, 16)]
    w2y = w2b[pl.ds(16, 16)]
    w2z = w2b[pl.ds(32, 16)]
    cols = [jnp.full((16,), c, _i32) for c in range(6)]

    base_pair = wid * 97 + jnp.minimum(wid, 21)
    npairs = 97 + (wid < 21).astype(_i32)

    sets = (
        (xb0, yb0, zb0, sidx0, didx0, fpos0, fneg0, vbuf0, in_sem0, sc_sem0),
        (xb1, yb1, zb1, sidx1, didx1, fpos1, fneg1, vbuf1, in_sem1, sc_sem1),
    )

    def issue_inputs(c, p):
        xb, yb, zb, sidx, didx, _, _, _, in_sem, _ = sets[p]
        e0 = c * CHUNK
        out = [
            pltpu.async_copy(evx.at[pl.ds(e0, CHUNK)], xb, in_sem),
            pltpu.async_copy(evy.at[pl.ds(e0, CHUNK)], yb, in_sem),
            pltpu.async_copy(evz.at[pl.ds(e0, CHUNK)], zb, in_sem),
        ]
        b0 = c * (CHUNK // 128)
        for j in range(4):
            out.append(pltpu.async_copy(
                eidx.at[b0 + j].at[0], sidx.at[j], in_sem))
            out.append(pltpu.async_copy(
                eidx.at[b0 + j].at[1], didx.at[j], in_sem))
        return out

    def compute(p):
        xb, yb, zb, _, _, fpos, fneg, vbuf, _, _ = sets[p]

        def gbody(g, carry):
            rows = iota + g * 16
            x = xb[pl.ds(g * 16, 16)]
            y = yb[pl.ds(g * 16, 16)]
            z = zb[pl.ds(g * 16, 16)]
            s = x * x * w2x + y * y * w2y + z * z * w2z
            m = jnp.exp(-s) * (-2.0)
            fx = m * w2x * x
            fy = m * w2y * y
            fz = m * w2z * z
            plsc.store_scatter(fpos, [rows, cols[0]], fx)
            plsc.store_scatter(fpos, [rows, cols[1]], fy)
            plsc.store_scatter(fpos, [rows, cols[2]], fz)
            plsc.store_scatter(fneg, [rows, cols[0]], -fx)
            plsc.store_scatter(fneg, [rows, cols[1]], -fy)
            plsc.store_scatter(fneg, [rows, cols[2]], -fz)
            plsc.store_scatter(vbuf, [rows, cols[0]], x * fx)
            plsc.store_scatter(vbuf, [rows, cols[1]], y * fy)
            plsc.store_scatter(vbuf, [rows, cols[2]], z * fz)
            plsc.store_scatter(vbuf, [rows, cols[3]], x * fy)
            plsc.store_scatter(vbuf, [rows, cols[4]], y * fz)
            plsc.store_scatter(vbuf, [rows, cols[5]], z * fx)
            return carry

        lax.fori_loop(0, GROUPS, gbody, 0)

    def fire_scatters(p):
        _, _, _, sidx, didx, fpos, fneg, vbuf, _, sc_sem = sets[p]
        for j in range(4):
            rs = pl.ds(j * 128, 128)
            pltpu.async_copy(fpos.at[rs], facc.at[sidx.at[j]], sc_sem,
                             add=True)
            pltpu.async_copy(fneg.at[rs], facc.at[didx.at[j]], sc_sem,
                             add=True)
            pltpu.async_copy(vbuf.at[rs], vacc.at[didx.at[j]], sc_sem,
                             add=True)

    def drain_scatters(p):
        _, _, _, sidx, didx, fpos, fneg, vbuf, _, sc_sem = sets[p]
        for j in range(4):
            rs = pl.ds(j * 128, 128)
            pltpu.make_async_copy(
                fpos.at[rs], facc.at[sidx.at[j]], sc_sem).wait()
            pltpu.make_async_copy(
                fneg.at[rs], facc.at[didx.at[j]], sc_sem).wait()
            pltpu.make_async_copy(
                vbuf.at[rs], vacc.at[didx.at[j]], sc_sem).wait()

    def pair_body(q, carry):
        c0 = 2 * (base_pair + q)
        @pl.when(q > 0)
        def _():
            drain_scatters(0)
        in0 = issue_inputs(c0, 0)
        @pl.when(q > 0)
        def _():
            drain_scatters(1)
        in1 = issue_inputs(c0 + 1, 1)
        for d in in0:
            d.wait()
        compute(0)
        fire_scatters(0)
        for d in in1:
            d.wait()
        compute(1)
        fire_scatters(1)
        return carry

    lax.fori_loop(0, npairs, pair_body, 0)

    drain_scatters(0)
    drain_scatters(1)
    plsc.subcore_barrier()

    rows = pl.ds(r0, ROWS_T)
    pltpu.sync_copy(facc.at[rows], f_out.at[cid].at[rows])
    pltpu.sync_copy(vacc.at[rows], v_out.at[cid].at[rows])


@jax.jit
def _sc_scatter(evx, evy, evz, eidx, w2r, z8r):
    mesh = plsc.VectorSubcoreMesh(core_axis_name="c", subcore_axis_name="s")
    return pl.kernel(
        _sc_body,
        out_type=(
            jax.ShapeDtypeStruct((2, NP, 8), _f32),
            jax.ShapeDtypeStruct((2, NP, 8), _f32),
        ),
        mesh=mesh,
        scratch_types=(
            [pltpu.VMEM((CHUNK,), _f32)] * 3
            + [pltpu.VMEM((4, 128), _i32)] * 2
            + [pltpu.VMEM((CHUNK, 8), _f32)] * 3
            + [pltpu.VMEM((CHUNK,), _f32)] * 3
            + [pltpu.VMEM((4, 128), _i32)] * 2
            + [pltpu.VMEM((CHUNK, 8), _f32)] * 3
            + [
                pltpu.VMEM((48,), _f32),
                pltpu.VMEM_SHARED((NP, 8), _f32),
                pltpu.VMEM_SHARED((NP, 8), _f32),
                pltpu.SemaphoreType.DMA,
                pltpu.SemaphoreType.DMA,
                pltpu.SemaphoreType.DMA,
                pltpu.SemaphoreType.DMA,
            ]
        ),
        compiler_params=pltpu.CompilerParams(
            needs_layout_passes=False, use_tc_tiling_on_sc=False),
    )(evx, evy, evz, eidx, w2r, z8r)


BN = 4000
NSTEP = N // BN


def _tc_body(num_ref, vol_ref, batch_ref, f_ref, v_ref,
             force_ref, stress_ref, acc_ref):
    i = pl.program_id(0)
    one = (num_ref[0] - N + 1).astype(_f32)
    force_ref[...] = (f_ref[0, :, :3] + f_ref[1, :, :3]) * one
    v = v_ref[0, :, :6] + v_ref[1, :, :6]
    b = batch_ref[0]
    bid = lax.broadcasted_iota(_i32, (NB, 1), 0)
    onehot = (b == bid).astype(_f32)
    contrib = lax.dot_general(onehot, v, (((1,), (0,)), ((), ())),
                              preferred_element_type=_f32)

    @pl.when(i == 0)
    def _():
        acc_ref[...] = jnp.zeros((NB, 6), _f32)

    acc_ref[...] += contrib

    @pl.when(i == NSTEP - 1)
    def _():
        stress_ref[...] = -acc_ref[...] / vol_ref[...]


@jax.jit
def _tc_finalize(num_atoms, vol_r, batch_r, fpart, vpart):
    return pl.pallas_call(
        _tc_body,
        grid=(NSTEP,),
        in_specs=[
            pl.BlockSpec(memory_space=pltpu.SMEM),
            pl.BlockSpec((NB, 1), lambda i: (0, 0)),
            pl.BlockSpec((1, 1, BN), lambda i: (i, 0, 0)),
            pl.BlockSpec((2, BN, 8), lambda i: (0, i, 0)),
            pl.BlockSpec((2, BN, 8), lambda i: (0, i, 0)),
        ],
        out_specs=[
            pl.BlockSpec((BN, 3), lambda i: (i, 0)),
            pl.BlockSpec((NB, 6), lambda i: (0, 0)),
        ],
        out_shape=[
            jax.ShapeDtypeStruct((N, 3), _f32),
            jax.ShapeDtypeStruct((NB, 6), _f32),
        ],
        scratch_shapes=[pltpu.VMEM((NB, 6), _f32)],
        compiler_params=pltpu.CompilerParams(
            dimension_semantics=("arbitrary",)),
    )(num_atoms, vol_r, batch_r, fpart, vpart)


def kernel(edge_vec, edge_idx, num_atoms, batch, cell_volume, W):
    evx = edge_vec[:, 0]
    evy = edge_vec[:, 1]
    evz = edge_vec[:, 2]
    eidx_r = edge_idx.reshape(2, E // 128, 128).transpose(1, 0, 2)
    w2r = jnp.broadcast_to((W * W)[:, None], (3, 16)).reshape(48)
    z8r = jnp.zeros((ROWS_T, 8), _f32)
    fpart, vpart = _sc_scatter(evx, evy, evz, eidx_r, w2r, z8r)
    batch_r = batch.reshape(NSTEP, 1, BN)
    vol_r = cell_volume.reshape(NB, 1)
    force, stress = _tc_finalize(num_atoms, vol_r, batch_r, fpart, vpart)
    return force, stress


# speedup vs baseline: 1.9681x; 1.0005x over previous
import jax
import jax.numpy as jnp
from jax import lax
from jax.experimental import pallas as pl
from jax.experimental.pallas import tpu as pltpu
from jax.experimental.pallas import tpu_sc as plsc

E = 3200000
N = 100000
NB = 4
NP = 100096
CHUNK = 512
NCHUNK = E // CHUNK
GROUPS = CHUNK // 16
ROWS_T = NP // 16

_f32 = jnp.float32
_i32 = jnp.int32


def _sc_body(evx, evy, evz, eidx, w2r, z8r, f_out, v_out,
             xb0, yb0, zb0, sidx0, didx0, fpos0, fneg0, vbuf0,
             xb1, yb1, zb1, sidx1, didx1, fpos1, fneg1, vbuf1,
             w2b, facc, vacc, in_sem0, in_sem1, sc_sem0, sc_sem1):
    cid = lax.axis_index("c")
    sid = lax.axis_index("s")
    wid = sid * 2 + cid

    r0 = sid * ROWS_T
    pltpu.sync_copy(z8r, facc.at[pl.ds(r0, ROWS_T)])
    pltpu.sync_copy(z8r, vacc.at[pl.ds(r0, ROWS_T)])
    pltpu.sync_copy(w2r, w2b)
    plsc.subcore_barrier()

    iota = lax.iota(_i32, 16)
    w2x = w2b[pl.ds(0, 16)]
    w2y = w2b[pl.ds(16, 16)]
    w2z = w2b[pl.ds(32, 16)]
    cols = [jnp.full((16,), c, _i32) for c in range(6)]

    base_pair = wid * 97 + jnp.minimum(wid, 21)
    npairs = 97 + (wid < 21).astype(_i32)

    sets = (
        (xb0, yb0, zb0, sidx0, didx0, fpos0, fneg0, vbuf0, in_sem0, sc_sem0),
        (xb1, yb1, zb1, sidx1, didx1, fpos1, fneg1, vbuf1, in_sem1, sc_sem1),
    )

    def issue_inputs(c, p):
        xb, yb, zb, sidx, didx, _, _, _, in_sem, _ = sets[p]
        e0 = c * CHUNK
        out = [
            pltpu.async_copy(evx.at[pl.ds(e0, CHUNK)], xb, in_sem),
            pltpu.async_copy(evy.at[pl.ds(e0, CHUNK)], yb, in_sem),
            pltpu.async_copy(evz.at[pl.ds(e0, CHUNK)], zb, in_sem),
        ]
        b0 = c * (CHUNK // 128)
        for j in range(4):
            out.append(pltpu.async_copy(
                eidx.at[b0 + j].at[0], sidx.at[j], in_sem))
            out.append(pltpu.async_copy(
                eidx.at[b0 + j].at[1], didx.at[j], in_sem))
        return out

    def compute(p):
        xb, yb, zb, _, _, fpos, fneg, vbuf, _, _ = sets[p]

        def gbody(g, carry):
          for dg in range(2):
            rows = iota + (g * 32 + dg * 16)
            x = xb[pl.ds(g * 32 + dg * 16, 16)]
            y = yb[pl.ds(g * 32 + dg * 16, 16)]
            z = zb[pl.ds(g * 32 + dg * 16, 16)]
            s = x * x * w2x + y * y * w2y + z * z * w2z
            m = jnp.exp(-s) * (-2.0)
            fx = m * w2x * x
            fy = m * w2y * y
            fz = m * w2z * z
            plsc.store_scatter(fpos, [rows, cols[0]], fx)
            plsc.store_scatter(fpos, [rows, cols[1]], fy)
            plsc.store_scatter(fpos, [rows, cols[2]], fz)
            plsc.store_scatter(fneg, [rows, cols[0]], -fx)
            plsc.store_scatter(fneg, [rows, cols[1]], -fy)
            plsc.store_scatter(fneg, [rows, cols[2]], -fz)
            plsc.store_scatter(vbuf, [rows, cols[0]], x * fx)
            plsc.store_scatter(vbuf, [rows, cols[1]], y * fy)
            plsc.store_scatter(vbuf, [rows, cols[2]], z * fz)
            plsc.store_scatter(vbuf, [rows, cols[3]], x * fy)
            plsc.store_scatter(vbuf, [rows, cols[4]], y * fz)
            plsc.store_scatter(vbuf, [rows, cols[5]], z * fx)
          return carry

        lax.fori_loop(0, GROUPS // 2, gbody, 0)

    def fire_scatters(p):
        _, _, _, sidx, didx, fpos, fneg, vbuf, _, sc_sem = sets[p]
        for j in range(4):
            rs = pl.ds(j * 128, 128)
            pltpu.async_copy(fpos.at[rs], facc.at[sidx.at[j]], sc_sem,
                             add=True)
            pltpu.async_copy(fneg.at[rs], facc.at[didx.at[j]], sc_sem,
                             add=True)
            pltpu.async_copy(vbuf.at[rs], vacc.at[didx.at[j]], sc_sem,
                             add=True)

    def drain_scatters(p):
        _, _, _, sidx, didx, fpos, fneg, vbuf, _, sc_sem = sets[p]
        for j in range(4):
            rs = pl.ds(j * 128, 128)
            pltpu.make_async_copy(
                fpos.at[rs], facc.at[sidx.at[j]], sc_sem).wait()
            pltpu.make_async_copy(
                fneg.at[rs], facc.at[didx.at[j]], sc_sem).wait()
            pltpu.make_async_copy(
                vbuf.at[rs], vacc.at[didx.at[j]], sc_sem).wait()

    def pair_body(q, carry):
        c0 = 2 * (base_pair + q)
        @pl.when(q > 0)
        def _():
            drain_scatters(0)
        in0 = issue_inputs(c0, 0)
        @pl.when(q > 0)
        def _():
            drain_scatters(1)
        in1 = issue_inputs(c0 + 1, 1)
        for d in in0:
            d.wait()
        compute(0)
        fire_scatters(0)
        for d in in1:
            d.wait()
        compute(1)
        fire_scatters(1)
        return carry

    lax.fori_loop(0, npairs, pair_body, 0)

    drain_scatters(0)
    drain_scatters(1)
    plsc.subcore_barrier()

    rows = pl.ds(r0, ROWS_T)
    pltpu.sync_copy(facc.at[rows], f_out.at[cid].at[rows])
    pltpu.sync_copy(vacc.at[rows], v_out.at[cid].at[rows])


@jax.jit
def _sc_scatter(evx, evy, evz, eidx, w2r, z8r):
    mesh = plsc.VectorSubcoreMesh(core_axis_name="c", subcore_axis_name="s")
    return pl.kernel(
        _sc_body,
        out_type=(
            jax.ShapeDtypeStruct((2, NP, 8), _f32),
            jax.ShapeDtypeStruct((2, NP, 8), _f32),
        ),
        mesh=mesh,
        scratch_types=(
            [pltpu.VMEM((CHUNK,), _f32)] * 3
            + [pltpu.VMEM((4, 128), _i32)] * 2
            + [pltpu.VMEM((CHUNK, 8), _f32)] * 3
            + [pltpu.VMEM((CHUNK,), _f32)] * 3
            + [pltpu.VMEM((4, 128), _i32)] * 2
            + [pltpu.VMEM((CHUNK, 8), _f32)] * 3
            + [
                pltpu.VMEM((48,), _f32),
                pltpu.VMEM_SHARED((NP, 8), _f32),
                pltpu.VMEM_SHARED((NP, 8), _f32),
                pltpu.SemaphoreType.DMA,
                pltpu.SemaphoreType.DMA,
                pltpu.SemaphoreType.DMA,
                pltpu.SemaphoreType.DMA,
            ]
        ),
        compiler_params=pltpu.CompilerParams(
            needs_layout_passes=False, use_tc_tiling_on_sc=False),
    )(evx, evy, evz, eidx, w2r, z8r)


BN = 4000
NSTEP = N // BN


def _tc_body(num_ref, vol_ref, batch_ref, f_ref, v_ref,
             force_ref, stress_ref, acc_ref):
    i = pl.program_id(0)
    one = (num_ref[0] - N + 1).astype(_f32)
    force_ref[...] = (f_ref[0, :, :3] + f_ref[1, :, :3]) * one
    v = v_ref[0, :, :6] + v_ref[1, :, :6]
    b = batch_ref[0]
    bid = lax.broadcasted_iota(_i32, (NB, 1), 0)
    onehot = (b == bid).astype(_f32)
    contrib = lax.dot_general(onehot, v, (((1,), (0,)), ((), ())),
                              preferred_element_type=_f32)

    @pl.when(i == 0)
    def _():
        acc_ref[...] = jnp.zeros((NB, 6), _f32)

    acc_ref[...] += contrib

    @pl.when(i == NSTEP - 1)
    def _():
        stress_ref[...] = -acc_ref[...] / vol_ref[...]


@jax.jit
def _tc_finalize(num_atoms, vol_r, batch_r, fpart, vpart):
    return pl.pallas_call(
        _tc_body,
        grid=(NSTEP,),
        in_specs=[
            pl.BlockSpec(memory_space=pltpu.SMEM),
            pl.BlockSpec((NB, 1), lambda i: (0, 0)),
            pl.BlockSpec((1, 1, BN), lambda i: (i, 0, 0)),
            pl.BlockSpec((2, BN, 8), lambda i: (0, i, 0)),
            pl.BlockSpec((2, BN, 8), lambda i: (0, i, 0)),
        ],
        out_specs=[
            pl.BlockSpec((BN, 3), lambda i: (i, 0)),
            pl.BlockSpec((NB, 6), lambda i: (0, 0)),
        ],
        out_shape=[
            jax.ShapeDtypeStruct((N, 3), _f32),
            jax.ShapeDtypeStruct((NB, 6), _f32),
        ],
        scratch_shapes=[pltpu.VMEM((NB, 6), _f32)],
        compiler_params=pltpu.CompilerParams(
            dimension_semantics=("arbitrary",)),
    )(num_atoms, vol_r, batch_r, fpart, vpart)


def kernel(edge_vec, edge_idx, num_atoms, batch, cell_volume, W):
    evx = edge_vec[:, 0]
    evy = edge_vec[:, 1]
    evz = edge_vec[:, 2]
    eidx_r = edge_idx.reshape(2, E // 128, 128).transpose(1, 0, 2)
    w2r = jnp.broadcast_to((W * W)[:, None], (3, 16)).reshape(48)
    z8r = jnp.zeros((ROWS_T, 8), _f32)
    fpart, vpart = _sc_scatter(evx, evy, evz, eidx_r, w2r, z8r)
    batch_r = batch.reshape(NSTEP, 1, BN)
    vol_r = cell_volume.reshape(NB, 1)
    force, stress = _tc_finalize(num_atoms, vol_r, batch_r, fpart, vpart)
    return force, stress
